# Initial kernel scaffold; baseline (speedup 1.0000x reference)
import jax
import jax.numpy as jnp
from jax.experimental import pallas as pl


def kernel(x, edge_index, W0, b0, W1, b1, W2, b2, actor_W, actor_b, critic_W, critic_b):
    raise NotImplementedError("write your pallas kernel here")


# speedup vs baseline: 5.6223x; 5.6223x over previous
import jax
import jax.numpy as jnp
from jax import lax
from jax.experimental import pallas as pl
from jax.experimental.pallas import tpu as pltpu
from jax.experimental.pallas import tpu_sc as plsc

N = 10000
E = 160000
D = 256
H = 256
A = 8

NC = 2
NS = 16
CH = 128
E_PAD = 163840
NCHUNK = E_PAD // CH
CPS = NCHUNK // NS
CPT = NCHUNK // (NC * NS)
N_PAD = 10112
ROWS_T = N_PAD // NS
HB = 128
RB = 632
GRID_R = N_PAD // RB

_mesh = plsc.VectorSubcoreMesh(core_axis_name="c", subcore_axis_name="s")


def _i32(*vals):
    return tuple(jnp.int32(v) for v in vals)


def _deg_body(dst2d, zrows, orows, out, hist_s, didx, ones_v):
    c = lax.axis_index("c")
    s = lax.axis_index("s")
    wid = s * NC + c
    pltpu.sync_copy(zrows.at[pl.ds(s * ROWS_T, ROWS_T)],
                    hist_s.at[pl.ds(s * ROWS_T, ROWS_T)])
    pltpu.sync_copy(orows, ones_v)
    pltpu.sync_copy(dst2d.at[pl.ds(wid * CPT, CPT)], didx)
    plsc.subcore_barrier()

    def body(j, carry):
        pltpu.sync_copy(ones_v, hist_s.at[didx.at[j]], add=True)
        return carry

    lax.fori_loop(jnp.int32(0), jnp.int32(CPT), body, jnp.int32(0))
    plsc.subcore_barrier()
    pltpu.sync_copy(hist_s.at[pl.ds(s * ROWS_T, ROWS_T)],
                    out.at[pl.ds(c * N_PAD + s * ROWS_T, ROWS_T)])


_deg = pl.kernel(
    _deg_body,
    out_type=jax.ShapeDtypeStruct((2 * N_PAD, HB), jnp.float32),
    mesh=_mesh,
    scratch_types=[
        pltpu.VMEM_SHARED((N_PAD, HB), jnp.float32),
        pltpu.VMEM((CPT, CH), jnp.int32),
        pltpu.VMEM((CH, HB), jnp.float32),
    ],
)


def _agg_body(g, srcall, dst2d, out, acc_s, sidx, didx, rows, sem):
    c = lax.axis_index("c")
    s = lax.axis_index("s")
    pltpu.sync_copy(g.at[pl.ds(c * N_PAD + s * ROWS_T, ROWS_T)],
                    acc_s.at[pl.ds(s * ROWS_T, ROWS_T)])
    pltpu.sync_copy(srcall.at[pl.ds(c * NCHUNK + s * CPS, CPS)], sidx)
    pltpu.sync_copy(dst2d.at[pl.ds(s * CPS, CPS)], didx)
    plsc.subcore_barrier()

    def body(j, carry):
        pltpu.async_copy(g.at[sidx.at[j]], rows, sem).wait()
        pltpu.sync_copy(rows, acc_s.at[didx.at[j]], add=True)
        return carry

    lax.fori_loop(jnp.int32(0), jnp.int32(CPS), body, jnp.int32(0))
    plsc.subcore_barrier()
    pltpu.sync_copy(acc_s.at[pl.ds(s * ROWS_T, ROWS_T)],
                    out.at[pl.ds(c * N_PAD + s * ROWS_T, ROWS_T)])


_agg = pl.kernel(
    _agg_body,
    out_type=jax.ShapeDtypeStruct((2 * N_PAD, 128), jnp.float32),
    mesh=_mesh,
    scratch_types=[
        pltpu.VMEM_SHARED((N_PAD, 128), jnp.float32),
        pltpu.VMEM((CPS, CH), jnp.int32),
        pltpu.VMEM((CPS, CH), jnp.int32),
        pltpu.VMEM((CH, 128), jnp.float32),
        pltpu.SemaphoreType.DMA,
    ],
)


def _dis(hlo, hhi):
    deg = hlo[:, 0:1] + hhi[:, 0:1] + 1.0
    return lax.rsqrt(deg)


def _dense0_body(x_ref, hlo, hhi, w_ref, out_ref):
    dis = _dis(hlo, hhi)
    g = jnp.dot(x_ref[...], w_ref[...], preferred_element_type=jnp.float32)
    out_ref[...] = g * dis


_dense0 = pl.pallas_call(
    _dense0_body,
    grid=(GRID_R, NC),
    in_specs=[
        pl.BlockSpec((RB, D), lambda i, c: _i32(i, 0)),
        pl.BlockSpec((RB, HB), lambda i, c: _i32(i, 0)),
        pl.BlockSpec((RB, HB), lambda i, c: _i32(GRID_R + i, 0)),
        pl.BlockSpec((D, 128), lambda i, c: _i32(0, c)),
    ],
    out_specs=pl.BlockSpec((RB, 128), lambda i, c: _i32(c * GRID_R + i, 0)),
    out_shape=jax.ShapeDtypeStruct((2 * N_PAD, 128), jnp.float32),
)


def _densem_body(alo, ahi, hlo, hhi, b_ref, w_ref, out_ref):
    dis = _dis(hlo, hhi)
    acc = jnp.concatenate([alo[...], ahi[...]], axis=1)
    h = jnp.maximum(acc * dis + b_ref[...], 0.0)
    g = jnp.dot(h, w_ref[...], preferred_element_type=jnp.float32)
    out_ref[...] = g * dis


_densem = pl.pallas_call(
    _densem_body,
    grid=(GRID_R, NC),
    in_specs=[
        pl.BlockSpec((RB, 128), lambda i, c: _i32(i, 0)),
        pl.BlockSpec((RB, 128), lambda i, c: _i32(GRID_R + i, 0)),
        pl.BlockSpec((RB, HB), lambda i, c: _i32(i, 0)),
        pl.BlockSpec((RB, HB), lambda i, c: _i32(GRID_R + i, 0)),
        pl.BlockSpec((1, H), lambda i, c: _i32(0, 0)),
        pl.BlockSpec((H, 128), lambda i, c: _i32(0, c)),
    ],
    out_specs=pl.BlockSpec((RB, 128), lambda i, c: _i32(c * GRID_R + i, 0)),
    out_shape=jax.ShapeDtypeStruct((2 * N_PAD, 128), jnp.float32),
)


def _read_body(*refs):
    alo = refs[0:A]
    ahi = refs[A:2 * A]
    hlo = refs[2 * A:3 * A]
    hhi = refs[3 * A:4 * A]
    b_ref, aw, ab, cw, cb, lo_ref, vo_ref = refs[4 * A:]
    for a in range(A):
        dis = _dis(hlo[a], hhi[a])
        acc = jnp.concatenate([alo[a][...], ahi[a][...]], axis=1)
        h = jnp.maximum(acc * dis + b_ref[...], 0.0)
        m = jnp.mean(h, axis=0, keepdims=True)
        mx = jnp.max(h, axis=0, keepdims=True)
        emb = jnp.concatenate([m, mx], axis=1)
        lo_ref[a:a + 1, :] = (
            jnp.dot(emb, aw[a], preferred_element_type=jnp.float32)
            + ab[a:a + 1, :])
        vo_ref[a:a + 1, :] = (jnp.sum(emb[0] * cw[a]).reshape(1, 1)
                              + cb[a:a + 1, :])


_AB = 125
_HI = N_PAD // 8

_read = pl.pallas_call(
    _read_body,
    grid=(1,),
    in_specs=(
        [pl.BlockSpec((8, 128), lambda i, a=a: _i32(_AB * a, 0)) for a in range(A)]
        + [pl.BlockSpec((8, 128), lambda i, a=a: _i32(_HI + _AB * a, 0))
           for a in range(A)]
        + [pl.BlockSpec((8, HB), lambda i, a=a: _i32(_AB * a, 0)) for a in range(A)]
        + [pl.BlockSpec((8, HB), lambda i, a=a: _i32(_HI + _AB * a, 0))
           for a in range(A)]
        + [
            pl.BlockSpec((1, H), lambda i: _i32(0, 0)),
            pl.BlockSpec((A, 2 * H, 16), lambda i: _i32(0, 0, 0)),
            pl.BlockSpec((A, 16), lambda i: _i32(0, 0)),
            pl.BlockSpec((A, 2 * H), lambda i: _i32(0, 0)),
            pl.BlockSpec((A, 1), lambda i: _i32(0, 0)),
        ]
    ),
    out_specs=[
        pl.BlockSpec((A, 16), lambda i: _i32(0, 0)),
        pl.BlockSpec((A, 1), lambda i: _i32(0, 0)),
    ],
    out_shape=[
        jax.ShapeDtypeStruct((A, 16), jnp.float32),
        jax.ShapeDtypeStruct((A, 1), jnp.float32),
    ],
)


def kernel(x, edge_index, W0, b0, W1, b1, W2, b2,
           actor_W, actor_b, critic_W, critic_b):
    x = jnp.pad(x.astype(jnp.float32), ((0, N_PAD - N), (0, 0)))
    ei = edge_index.astype(jnp.int32)
    pad = E_PAD - E
    src = jnp.concatenate([ei[0], jnp.zeros((pad,), jnp.int32)])
    dst = jnp.concatenate([ei[1], jnp.full((pad,), N, jnp.int32)])
    src2d = src.reshape(NCHUNK, CH)
    dst2d = dst.reshape(NCHUNK, CH)
    srchi = src2d + N_PAD
    zrows = jnp.zeros((N_PAD, HB), jnp.float32)
    orows = jnp.ones((CH, HB), jnp.float32)

    srcall = jnp.concatenate([src2d, srchi])
    hist = _deg(dst2d, zrows, orows)
    g0 = _dense0(x, hist, hist, W0)
    acc0 = _agg(g0, srcall, dst2d)
    g1 = _densem(acc0, acc0, hist, hist, b0.reshape(1, H), W1)
    acc1 = _agg(g1, srcall, dst2d)
    g2 = _densem(acc1, acc1, hist, hist, b1.reshape(1, H), W2)
    acc2 = _agg(g2, srcall, dst2d)
    read_args = ([acc2] * A + [acc2] * A + [hist] * A + [hist] * A
                 + [b2.reshape(1, H), actor_W, actor_b,
                    critic_W[..., 0], critic_b])
    logits, values = _read(*read_args)
    return logits, values
